# baseline (speedup 1.0000x reference)
import functools

import jax
import jax.numpy as jnp
from jax import lax
from jax.experimental import pallas as pl
from jax.experimental.pallas import tpu as pltpu
from jax.experimental.pallas import tpu_sc as plsc

N = 10000
D = 128
O_DIM = 40
E = 320000
NW = 32
EPB = 128
NB = 84
E_PAD = NW * NB * EPB
NP = 10008
RPT = 632
RPT_LAST = NP - 15 * RPT
BLK = 1000
GRID = N // BLK

_HIGH = jax.lax.Precision.HIGHEST


def _dot(a, b):
    return jnp.dot(a, b, precision=_HIGH, preferred_element_type=jnp.float32)


def _sc_mesh():
    return plsc.VectorSubcoreMesh(core_axis_name="c", subcore_axis_name="s",
                                  num_cores=2, num_subcores=16)


def _zero_slice(zeros, accum, s):
    @pl.when(s < 15)
    def _():
        pltpu.sync_copy(zeros.at[pl.ds(s * RPT, RPT)],
                        accum.at[pl.ds(s * RPT, RPT)])

    @pl.when(s == 15)
    def _():
        pltpu.sync_copy(zeros.at[pl.ds(15 * RPT, RPT_LAST)],
                        accum.at[pl.ds(15 * RPT, RPT_LAST)])


def _drain_slice(accum, out, c, s):
    @pl.when(s < 15)
    def _():
        pltpu.sync_copy(accum.at[pl.ds(s * RPT, RPT)],
                        out.at[c, pl.ds(s * RPT, RPT)])

    @pl.when(s == 15)
    def _():
        pltpu.sync_copy(accum.at[pl.ds(15 * RPT, RPT_LAST)],
                        out.at[c, pl.ds(15 * RPT, RPT_LAST)])


@functools.lru_cache(maxsize=None)
def _make_seg_scatter():
    @functools.partial(
        pl.kernel,
        out_type=jax.ShapeDtypeStruct((2, NP, D), jnp.float32),
        mesh=_sc_mesh(),
        scratch_types=[
            pltpu.VMEM((2, EPB), jnp.int32),
            pltpu.VMEM((2, EPB), jnp.int32),
            pltpu.VMEM((2, EPB), jnp.int32),
            pltpu.VMEM((2, EPB), jnp.int32),
            pltpu.VMEM((EPB, D), jnp.float32),
            pltpu.VMEM((EPB, D), jnp.float32),
            pltpu.VMEM_SHARED((NP, D), jnp.float32),
            pltpu.SemaphoreType.DMA,
            pltpu.SemaphoreType.DMA,
            pltpu.SemaphoreType.DMA,
            pltpu.SemaphoreType.DMA,
            pltpu.SemaphoreType.DMA,
            pltpu.SemaphoreType.DMA,
            pltpu.SemaphoreType.DMA,
            pltpu.SemaphoreType.DMA,
        ],
        compiler_params=pltpu.CompilerParams(use_tc_tiling_on_sc=False),
    )
    def seg(table, eidx, zeros, out, ib0, ib1, ib2, ib3, rows0, rows1,
            accum, is0, is1, is2, is3, g0s, g1s, s0s, s1s):
        c = lax.axis_index("c")
        s = lax.axis_index("s")
        wid = c * 16 + s
        ibs = [ib0, ib1, ib2, ib3]
        isems = [is0, is1, is2, is3]
        rowss = [rows0, rows1]
        gsems = [g0s, g1s]
        ssems = [s0s, s1s]

        def i_issue(k, m):
            pltpu.async_copy(eidx.at[wid, k], ibs[m], isems[m])

        def i_wait(k, m):
            pltpu.make_async_copy(eidx.at[wid, k], ibs[m], isems[m]).wait()

        def g_issue(m, r):
            return pltpu.async_copy(table.at[ibs[m].at[0]], rowss[r],
                                    gsems[r])

        def s_issue(m, r):
            return pltpu.async_copy(rowss[r], accum.at[ibs[m].at[1]],
                                    ssems[r], add=True)

        for m in range(4):
            i_issue(m, m)
        _zero_slice(zeros, accum, s)
        plsc.subcore_barrier()

        @pl.loop(0, NB, step=4)
        def _(j):
            for m in range(4):
                i_wait(j + m, m)
            g0 = g_issue(0, 0)
            g1 = g_issue(1, 1)
            g0.wait()
            s0 = s_issue(0, 0)
            g1.wait()
            s1 = s_issue(1, 1)
            s0.wait()
            g2 = g_issue(2, 0)
            s1.wait()
            g3 = g_issue(3, 1)
            g2.wait()
            s2 = s_issue(2, 0)
            g3.wait()
            s3 = s_issue(3, 1)
            s2.wait()
            s3.wait()

            @pl.when(j + 4 < NB)
            def _():
                for m in range(4):
                    i_issue(j + 4 + m, m)

        plsc.subcore_barrier()
        _drain_slice(accum, out, c, s)

    return seg


@functools.lru_cache(maxsize=None)
def _make_edge_count():
    @functools.partial(
        pl.kernel,
        out_type=jax.ShapeDtypeStruct((2, NP, 16), jnp.float32),
        mesh=_sc_mesh(),
        scratch_types=[
            pltpu.VMEM((NB, 2, EPB), jnp.int32),
            pltpu.VMEM((EPB, 16), jnp.float32),
            pltpu.VMEM_SHARED((NP, 16), jnp.float32),
            pltpu.SemaphoreType.DMA,
        ],
        compiler_params=pltpu.CompilerParams(use_tc_tiling_on_sc=False),
    )
    def cntk(eidx, zeros16, ones_hbm, out, eidx_v, ones_v, accum, s0):
        c = lax.axis_index("c")
        s = lax.axis_index("s")
        wid = c * 16 + s
        _zero_slice(zeros16, accum, s)
        pltpu.sync_copy(eidx.at[wid], eidx_v)
        pltpu.sync_copy(ones_hbm, ones_v)
        plsc.subcore_barrier()

        @pl.loop(0, NB)
        def _(j):
            pltpu.async_copy(ones_v, accum.at[eidx_v.at[j, 1]], s0,
                             add=True).wait()

        plsc.subcore_barrier()
        _drain_slice(accum, out, c, s)

    return cntk


def _tc_a_body(x_ref, w_ref, b_ref, o_ref):
    o_ref[...] = jnp.maximum(_dot(x_ref[...], w_ref[...]) + b_ref[...], 0.0)


def _tc_a(x, w, b):
    return pl.pallas_call(
        _tc_a_body,
        grid=(GRID,),
        in_specs=[
            pl.BlockSpec((BLK, D), lambda i: (i, 0)),
            pl.BlockSpec((D, D), lambda i: (0, 0)),
            pl.BlockSpec((1, D), lambda i: (0, 0)),
        ],
        out_specs=pl.BlockSpec((BLK, D), lambda i: (i, 0)),
        out_shape=jax.ShapeDtypeStruct((N, D), jnp.float32),
    )(x, w, b)


def _tc_b_body(x_ref, p_ref, c_ref, awx_ref, awm_ref, ab_ref, lw_ref, lb_ref,
               h1_ref, t_ref, inv_ref):
    p = p_ref[0] + p_ref[1]
    cnt = (c_ref[0] + c_ref[1])[:, 0:1]
    inv = 1.0 / jnp.maximum(cnt, 1.0)
    mean = p * inv
    h = jnp.maximum(_dot(x_ref[...], awx_ref[...])
                    + _dot(mean, awm_ref[...]) + ab_ref[...], 0.0)
    nrm = jnp.sqrt(jnp.sum(h * h, axis=1, keepdims=True))
    h1 = h / jnp.maximum(nrm, 1e-12)
    h1_ref[...] = h1
    t_ref[...] = jnp.maximum(_dot(h1, lw_ref[...]) + lb_ref[...], 0.0)
    inv_ref[...] = jnp.broadcast_to(inv, (BLK, D))


def _tc_b(x, partials, cnt_partials, awx, awm, ab, lw, lb):
    return pl.pallas_call(
        _tc_b_body,
        grid=(GRID,),
        in_specs=[
            pl.BlockSpec((BLK, D), lambda i: (i, 0)),
            pl.BlockSpec((2, BLK, D), lambda i: (0, i, 0)),
            pl.BlockSpec((2, BLK, 16), lambda i: (0, i, 0)),
            pl.BlockSpec((D, D), lambda i: (0, 0)),
            pl.BlockSpec((D, D), lambda i: (0, 0)),
            pl.BlockSpec((1, D), lambda i: (0, 0)),
            pl.BlockSpec((D, D), lambda i: (0, 0)),
            pl.BlockSpec((1, D), lambda i: (0, 0)),
        ],
        out_specs=[
            pl.BlockSpec((BLK, D), lambda i: (i, 0)),
            pl.BlockSpec((BLK, D), lambda i: (i, 0)),
            pl.BlockSpec((BLK, D), lambda i: (i, 0)),
        ],
        out_shape=[
            jax.ShapeDtypeStruct((N, D), jnp.float32),
            jax.ShapeDtypeStruct((N, D), jnp.float32),
            jax.ShapeDtypeStruct((N, D), jnp.float32),
        ],
    )(x, partials, cnt_partials, awx, awm, ab, lw, lb)


def _tc_c_body(h1_ref, p_ref, inv_ref, awx_ref, awm_ref, ab_ref,
               pw1_ref, pb1_ref, pw2_ref, pb2_ref, o_ref):
    mean = (p_ref[0] + p_ref[1]) * inv_ref[...]
    h = jnp.maximum(_dot(h1_ref[...], awx_ref[...])
                    + _dot(mean, awm_ref[...]) + ab_ref[...], 0.0)
    nrm = jnp.sqrt(jnp.sum(h * h, axis=1, keepdims=True))
    h2 = h / jnp.maximum(nrm, 1e-12)
    h3 = _dot(h2, pw1_ref[...]) + pb1_ref[...]
    z = _dot(h3, pw2_ref[...]) + pb2_ref[...]
    m = jnp.max(z, axis=1, keepdims=True)
    lse = m + jnp.log(jnp.sum(jnp.exp(z - m), axis=1, keepdims=True))
    o_ref[...] = (z - lse)[:, :O_DIM]


def _tc_c(h1, partials, inv, awx, awm, ab, pw1, pb1, pw2, pb2):
    return pl.pallas_call(
        _tc_c_body,
        grid=(GRID,),
        in_specs=[
            pl.BlockSpec((BLK, D), lambda i: (i, 0)),
            pl.BlockSpec((2, BLK, D), lambda i: (0, i, 0)),
            pl.BlockSpec((BLK, D), lambda i: (i, 0)),
            pl.BlockSpec((D, D), lambda i: (0, 0)),
            pl.BlockSpec((D, D), lambda i: (0, 0)),
            pl.BlockSpec((1, D), lambda i: (0, 0)),
            pl.BlockSpec((D, D), lambda i: (0, 0)),
            pl.BlockSpec((1, D), lambda i: (0, 0)),
            pl.BlockSpec((D, D), lambda i: (0, 0)),
            pl.BlockSpec((1, D), lambda i: (0, 0)),
        ],
        out_specs=pl.BlockSpec((BLK, O_DIM), lambda i: (i, 0)),
        out_shape=jax.ShapeDtypeStruct((N, O_DIM), jnp.float32),
    )(h1, partials, inv, awx, awm, ab, pw1, pb1, pw2, pb2)


def kernel(x, edge_index, lin_W0, lin_b0, agg_W0, agg_b0,
           lin_W1, lin_b1, agg_W1, agg_b1,
           post_W1, post_b1, post_W2, post_b2):
    src = edge_index[0].astype(jnp.int32)
    dst = edge_index[1].astype(jnp.int32)
    pad = E_PAD - E
    src3 = jnp.concatenate([src, jnp.zeros((pad,), jnp.int32)]
                           ).reshape(NW, NB, EPB)
    dst3 = jnp.concatenate([dst, N + (jnp.arange(pad, dtype=jnp.int32) % 8)]
                           ).reshape(NW, NB, EPB)
    eidx = jnp.stack([src3, dst3], axis=2)
    zeros128 = jnp.zeros((NP, D), jnp.float32)
    zeros16 = jnp.zeros((NP, 16), jnp.float32)
    ones16 = jnp.ones((EPB, 16), jnp.float32)

    lb0 = lin_b0.reshape(1, D)
    lb1 = lin_b1.reshape(1, D)
    ab0 = agg_b0.reshape(1, D)
    ab1 = agg_b1.reshape(1, D)
    pb1 = post_b1.reshape(1, D)
    pw2 = jnp.pad(post_W2, ((0, 0), (0, D - O_DIM)))
    pb2 = jnp.concatenate([post_b2,
                           jnp.full((D - O_DIM,), -1e30, jnp.float32)]
                          ).reshape(1, D)

    cntp = _make_edge_count()(eidx, zeros16, ones16)
    table0 = _tc_a(x, lin_W0, lb0)
    part0 = _make_seg_scatter()(table0, eidx, zeros128)
    h1, table1, inv = _tc_b(x, part0, cntp, agg_W0[:D], agg_W0[D:], ab0,
                            lin_W1, lb1)
    part1 = _make_seg_scatter()(table1, eidx, zeros128)
    return _tc_c(h1, part1, inv, agg_W1[:D], agg_W1[D:], ab1,
                 post_W1, pb1, pw2, pb2)


# speedup vs baseline: 1.1576x; 1.1576x over previous
import functools

import jax
import jax.numpy as jnp
from jax import lax
from jax.experimental import pallas as pl
from jax.experimental.pallas import tpu as pltpu
from jax.experimental.pallas import tpu_sc as plsc

N = 10000
D = 128
O_DIM = 40
E = 320000
NW = 32
EPB = 128
NB = 84
E_PAD = NW * NB * EPB
NP = 11136
RPT = NP // 16
RPT_LAST = RPT
BLK = 1000
GRID = N // BLK

_HIGH = jax.lax.Precision.HIGHEST


def _dot(a, b):
    return jnp.dot(a, b, precision=_HIGH, preferred_element_type=jnp.float32)


def _sc_mesh():
    return plsc.VectorSubcoreMesh(core_axis_name="c", subcore_axis_name="s",
                                  num_cores=2, num_subcores=16)


def _zero_slice(zeros, accum, s):
    pltpu.sync_copy(zeros.at[pl.ds(s * RPT, RPT)],
                    accum.at[pl.ds(s * RPT, RPT)])


def _drain_slice(accum, out, c, s):
    pltpu.sync_copy(accum.at[pl.ds(s * RPT, RPT)],
                    out.at[c, pl.ds(s * RPT, RPT)])


@functools.lru_cache(maxsize=None)
def _make_seg_scatter():
    @functools.partial(
        pl.kernel,
        out_type=jax.ShapeDtypeStruct((2, NP, D), jnp.float32),
        mesh=_sc_mesh(),
        scratch_types=[
            pltpu.VMEM((2, EPB), jnp.int32),
            pltpu.VMEM((2, EPB), jnp.int32),
            pltpu.VMEM((2, EPB), jnp.int32),
            pltpu.VMEM((2, EPB), jnp.int32),
            pltpu.VMEM((EPB, D), jnp.float32),
            pltpu.VMEM((EPB, D), jnp.float32),
            pltpu.VMEM_SHARED((NP, D), jnp.float32),
            pltpu.SemaphoreType.DMA,
            pltpu.SemaphoreType.DMA,
            pltpu.SemaphoreType.DMA,
            pltpu.SemaphoreType.DMA,
            pltpu.SemaphoreType.DMA,
            pltpu.SemaphoreType.DMA,
            pltpu.SemaphoreType.DMA,
            pltpu.SemaphoreType.DMA,
        ],
        compiler_params=pltpu.CompilerParams(use_tc_tiling_on_sc=False),
    )
    def seg(table, eidx, zeros, out, ib0, ib1, ib2, ib3, rows0, rows1,
            accum, is0, is1, is2, is3, g0s, g1s, s0s, s1s):
        c = lax.axis_index("c")
        s = lax.axis_index("s")
        wid = c * 16 + s
        ibs = [ib0, ib1, ib2, ib3]
        isems = [is0, is1, is2, is3]
        rowss = [rows0, rows1]
        gsems = [g0s, g1s]
        ssems = [s0s, s1s]

        def i_issue(k, m):
            pltpu.async_copy(eidx.at[wid, k], ibs[m], isems[m])

        def i_wait(k, m):
            pltpu.make_async_copy(eidx.at[wid, k], ibs[m], isems[m]).wait()

        def g_issue(m, r):
            return pltpu.async_copy(table.at[ibs[m].at[0]], rowss[r],
                                    gsems[r])

        def s_issue(m, r):
            return pltpu.async_copy(rowss[r], accum.at[ibs[m].at[1]],
                                    ssems[r], add=True)

        for m in range(4):
            i_issue(m, m)
        _zero_slice(zeros, accum, s)
        plsc.subcore_barrier()

        @pl.loop(0, NB, step=4)
        def _(j):
            for m in range(4):
                i_wait(j + m, m)
            g0 = g_issue(0, 0)
            g1 = g_issue(1, 1)
            g0.wait()
            s0 = s_issue(0, 0)
            g1.wait()
            s1 = s_issue(1, 1)
            s0.wait()
            g2 = g_issue(2, 0)
            s1.wait()
            g3 = g_issue(3, 1)
            g2.wait()
            s2 = s_issue(2, 0)
            g3.wait()
            s3 = s_issue(3, 1)
            s2.wait()
            s3.wait()

            @pl.when(j + 4 < NB)
            def _():
                for m in range(4):
                    i_issue(j + 4 + m, m)

        plsc.subcore_barrier()
        _drain_slice(accum, out, c, s)

    return seg


@functools.lru_cache(maxsize=None)
def _make_edge_count():
    @functools.partial(
        pl.kernel,
        out_type=jax.ShapeDtypeStruct((2, NP, 16), jnp.float32),
        mesh=_sc_mesh(),
        scratch_types=[
            pltpu.VMEM((NB, 2, EPB), jnp.int32),
            pltpu.VMEM((EPB, 16), jnp.float32),
            pltpu.VMEM_SHARED((NP, 16), jnp.float32),
            pltpu.SemaphoreType.DMA,
        ],
        compiler_params=pltpu.CompilerParams(use_tc_tiling_on_sc=False),
    )
    def cntk(eidx, zeros16, ones_hbm, out, eidx_v, ones_v, accum, s0):
        c = lax.axis_index("c")
        s = lax.axis_index("s")
        wid = c * 16 + s
        _zero_slice(zeros16, accum, s)
        pltpu.sync_copy(eidx.at[wid], eidx_v)
        pltpu.sync_copy(ones_hbm, ones_v)
        plsc.subcore_barrier()

        @pl.loop(0, NB)
        def _(j):
            pltpu.async_copy(ones_v, accum.at[eidx_v.at[j, 1]], s0,
                             add=True).wait()

        plsc.subcore_barrier()
        _drain_slice(accum, out, c, s)

    return cntk


def _tc_a_body(x_ref, w_ref, b_ref, o_ref):
    o_ref[...] = jnp.maximum(_dot(x_ref[...], w_ref[...]) + b_ref[...], 0.0)


def _tc_a(x, w, b):
    return pl.pallas_call(
        _tc_a_body,
        grid=(GRID,),
        in_specs=[
            pl.BlockSpec((BLK, D), lambda i: (i, 0)),
            pl.BlockSpec((D, D), lambda i: (0, 0)),
            pl.BlockSpec((1, D), lambda i: (0, 0)),
        ],
        out_specs=pl.BlockSpec((BLK, D), lambda i: (i, 0)),
        out_shape=jax.ShapeDtypeStruct((N, D), jnp.float32),
    )(x, w, b)


def _tc_b_body(x_ref, p_ref, c_ref, awx_ref, awm_ref, ab_ref, lw_ref, lb_ref,
               h1_ref, t_ref, inv_ref):
    p = p_ref[0] + p_ref[1]
    cnt = (c_ref[0] + c_ref[1])[:, 0:1]
    inv = 1.0 / jnp.maximum(cnt, 1.0)
    mean = p * inv
    h = jnp.maximum(_dot(x_ref[...], awx_ref[...])
                    + _dot(mean, awm_ref[...]) + ab_ref[...], 0.0)
    nrm = jnp.sqrt(jnp.sum(h * h, axis=1, keepdims=True))
    h1 = h / jnp.maximum(nrm, 1e-12)
    h1_ref[...] = h1
    t_ref[...] = jnp.maximum(_dot(h1, lw_ref[...]) + lb_ref[...], 0.0)
    inv_ref[...] = jnp.broadcast_to(inv, (BLK, D))


def _tc_b(x, partials, cnt_partials, awx, awm, ab, lw, lb):
    return pl.pallas_call(
        _tc_b_body,
        grid=(GRID,),
        in_specs=[
            pl.BlockSpec((BLK, D), lambda i: (i, 0)),
            pl.BlockSpec((2, BLK, D), lambda i: (0, i, 0)),
            pl.BlockSpec((2, BLK, 16), lambda i: (0, i, 0)),
            pl.BlockSpec((D, D), lambda i: (0, 0)),
            pl.BlockSpec((D, D), lambda i: (0, 0)),
            pl.BlockSpec((1, D), lambda i: (0, 0)),
            pl.BlockSpec((D, D), lambda i: (0, 0)),
            pl.BlockSpec((1, D), lambda i: (0, 0)),
        ],
        out_specs=[
            pl.BlockSpec((BLK, D), lambda i: (i, 0)),
            pl.BlockSpec((BLK, D), lambda i: (i, 0)),
            pl.BlockSpec((BLK, D), lambda i: (i, 0)),
        ],
        out_shape=[
            jax.ShapeDtypeStruct((N, D), jnp.float32),
            jax.ShapeDtypeStruct((N, D), jnp.float32),
            jax.ShapeDtypeStruct((N, D), jnp.float32),
        ],
    )(x, partials, cnt_partials, awx, awm, ab, lw, lb)


def _tc_c_body(h1_ref, p_ref, inv_ref, awx_ref, awm_ref, ab_ref,
               pw1_ref, pb1_ref, pw2_ref, pb2_ref, o_ref):
    mean = (p_ref[0] + p_ref[1]) * inv_ref[...]
    h = jnp.maximum(_dot(h1_ref[...], awx_ref[...])
                    + _dot(mean, awm_ref[...]) + ab_ref[...], 0.0)
    nrm = jnp.sqrt(jnp.sum(h * h, axis=1, keepdims=True))
    h2 = h / jnp.maximum(nrm, 1e-12)
    h3 = _dot(h2, pw1_ref[...]) + pb1_ref[...]
    z = _dot(h3, pw2_ref[...]) + pb2_ref[...]
    m = jnp.max(z, axis=1, keepdims=True)
    lse = m + jnp.log(jnp.sum(jnp.exp(z - m), axis=1, keepdims=True))
    o_ref[...] = (z - lse)[:, :O_DIM]


def _tc_c(h1, partials, inv, awx, awm, ab, pw1, pb1, pw2, pb2):
    return pl.pallas_call(
        _tc_c_body,
        grid=(GRID,),
        in_specs=[
            pl.BlockSpec((BLK, D), lambda i: (i, 0)),
            pl.BlockSpec((2, BLK, D), lambda i: (0, i, 0)),
            pl.BlockSpec((BLK, D), lambda i: (i, 0)),
            pl.BlockSpec((D, D), lambda i: (0, 0)),
            pl.BlockSpec((D, D), lambda i: (0, 0)),
            pl.BlockSpec((1, D), lambda i: (0, 0)),
            pl.BlockSpec((D, D), lambda i: (0, 0)),
            pl.BlockSpec((1, D), lambda i: (0, 0)),
            pl.BlockSpec((D, D), lambda i: (0, 0)),
            pl.BlockSpec((1, D), lambda i: (0, 0)),
        ],
        out_specs=pl.BlockSpec((BLK, O_DIM), lambda i: (i, 0)),
        out_shape=jax.ShapeDtypeStruct((N, O_DIM), jnp.float32),
    )(h1, partials, inv, awx, awm, ab, pw1, pb1, pw2, pb2)


def kernel(x, edge_index, lin_W0, lin_b0, agg_W0, agg_b0,
           lin_W1, lin_b1, agg_W1, agg_b1,
           post_W1, post_b1, post_W2, post_b2):
    src = edge_index[0].astype(jnp.int32)
    dst = edge_index[1].astype(jnp.int32)
    pad = E_PAD - E
    src3 = jnp.concatenate([src, jnp.zeros((pad,), jnp.int32)]
                           ).reshape(NW, NB, EPB)
    dst3 = jnp.concatenate(
        [dst, N + (jnp.arange(pad, dtype=jnp.int32) % (NP - N))]
    ).reshape(NW, NB, EPB)
    eidx = jnp.stack([src3, dst3], axis=2)
    zeros128 = jnp.zeros((NP, D), jnp.float32)
    zeros16 = jnp.zeros((NP, 16), jnp.float32)
    ones16 = jnp.ones((EPB, 16), jnp.float32)

    lb0 = lin_b0.reshape(1, D)
    lb1 = lin_b1.reshape(1, D)
    ab0 = agg_b0.reshape(1, D)
    ab1 = agg_b1.reshape(1, D)
    pb1 = post_b1.reshape(1, D)
    pw2 = jnp.pad(post_W2, ((0, 0), (0, D - O_DIM)))
    pb2 = jnp.concatenate([post_b2,
                           jnp.full((D - O_DIM,), -1e30, jnp.float32)]
                          ).reshape(1, D)

    cntp = _make_edge_count()(eidx, zeros16, ones16)
    table0 = _tc_a(x, lin_W0, lb0)
    part0 = _make_seg_scatter()(table0, eidx, zeros128)
    h1, table1, inv = _tc_b(x, part0, cntp, agg_W0[:D], agg_W0[D:], ab0,
                            lin_W1, lb1)
    part1 = _make_seg_scatter()(table1, eidx, zeros128)
    return _tc_c(h1, part1, inv, agg_W1[:D], agg_W1[D:], ab1,
                 post_W1, pb1, pw2, pb2)


# speedup vs baseline: 5.4806x; 4.7346x over previous
import functools

import jax
import jax.numpy as jnp
from jax import lax
from jax.experimental import pallas as pl
from jax.experimental.pallas import tpu as pltpu
from jax.experimental.pallas import tpu_sc as plsc

N = 10000
D = 128
O_DIM = 40
E = 320000
NW = 32
EPB = 128
NB = 84
E_PAD = NW * NB * EPB
NP = 11136
RPT = NP // 16
RPT_LAST = RPT
BLK = 1000
GRID = N // BLK

_HIGH = jax.lax.Precision.HIGHEST


def _dot(a, b):
    return jnp.dot(a, b, precision=_HIGH, preferred_element_type=jnp.float32)


def _sc_mesh():
    return plsc.VectorSubcoreMesh(core_axis_name="c", subcore_axis_name="s",
                                  num_cores=2, num_subcores=16)


def _zero_slice(zeros, accum, s):
    pltpu.sync_copy(zeros.at[pl.ds(s * RPT, RPT)],
                    accum.at[pl.ds(s * RPT, RPT)])


def _drain_slice(accum, out, c, s):
    pltpu.sync_copy(accum.at[pl.ds(s * RPT, RPT)],
                    out.at[c, pl.ds(s * RPT, RPT)])


@functools.lru_cache(maxsize=None)
def _make_seg_scatter():
    @functools.partial(
        pl.kernel,
        out_type=jax.ShapeDtypeStruct((2, NP, D), jnp.float32),
        mesh=_sc_mesh(),
        scratch_types=[
            pltpu.VMEM((2, EPB), jnp.int32),
            pltpu.VMEM((2, EPB), jnp.int32),
            pltpu.VMEM((2, EPB), jnp.int32),
            pltpu.VMEM((2, EPB), jnp.int32),
            pltpu.VMEM((EPB, D), jnp.float32),
            pltpu.VMEM((EPB, D), jnp.float32),
            pltpu.VMEM_SHARED((NP, D), jnp.float32),
            pltpu.SemaphoreType.DMA,
            pltpu.SemaphoreType.DMA,
            pltpu.SemaphoreType.DMA,
            pltpu.SemaphoreType.DMA,
            pltpu.SemaphoreType.DMA,
            pltpu.SemaphoreType.DMA,
            pltpu.SemaphoreType.DMA,
            pltpu.SemaphoreType.DMA,
        ],
        compiler_params=pltpu.CompilerParams(use_tc_tiling_on_sc=False),
    )
    def seg(table, eidx, zeros, out, ib0, ib1, ib2, ib3, rows0, rows1,
            accum, is0, is1, is2, is3, g0s, g1s, s0s, s1s):
        c = lax.axis_index("c")
        s = lax.axis_index("s")
        wid = c * 16 + s
        ibs = [ib0, ib1, ib2, ib3]
        isems = [is0, is1, is2, is3]
        rowss = [rows0, rows1]
        gsems = [g0s, g1s]
        ssems = [s0s, s1s]

        def i_issue(k, m):
            pltpu.async_copy(eidx.at[wid, k], ibs[m], isems[m])

        def i_wait(k, m):
            pltpu.make_async_copy(eidx.at[wid, k], ibs[m], isems[m]).wait()

        def g_issue(m, r):
            return pltpu.async_copy(table.at[ibs[m].at[0]], rowss[r],
                                    gsems[r])

        def s_issue(m, r):
            return pltpu.async_copy(rowss[r], accum.at[ibs[m].at[1]],
                                    ssems[r], add=True)

        for m in range(4):
            i_issue(m, m)
        _zero_slice(zeros, accum, s)
        plsc.subcore_barrier()

        @pl.loop(0, NB, step=4)
        def _(j):
            for m in range(4):
                i_wait(j + m, m)
            g0 = g_issue(0, 0)
            g1 = g_issue(1, 1)
            g0.wait()
            s0 = s_issue(0, 0)
            g1.wait()
            s1 = s_issue(1, 1)
            s0.wait()
            g2 = g_issue(2, 0)
            s1.wait()
            g3 = g_issue(3, 1)
            g2.wait()
            s2 = s_issue(2, 0)
            g3.wait()
            s3 = s_issue(3, 1)
            s2.wait()
            s3.wait()

            @pl.when(j + 4 < NB)
            def _():
                for m in range(4):
                    i_issue(j + 4 + m, m)

        plsc.subcore_barrier()
        _drain_slice(accum, out, c, s)

    return seg


@functools.lru_cache(maxsize=None)
def _make_edge_count():
    @functools.partial(
        pl.kernel,
        out_type=jax.ShapeDtypeStruct((2, NP, 16), jnp.float32),
        mesh=_sc_mesh(),
        scratch_types=[
            pltpu.VMEM((NB, 2, EPB), jnp.int32),
            pltpu.VMEM((EPB, 16), jnp.float32),
            pltpu.VMEM_SHARED((NP, 16), jnp.float32),
            pltpu.SemaphoreType.DMA,
        ],
        compiler_params=pltpu.CompilerParams(use_tc_tiling_on_sc=False),
    )
    def cntk(eidx, zeros16, ones_hbm, out, eidx_v, ones_v, accum, s0):
        c = lax.axis_index("c")
        s = lax.axis_index("s")
        wid = c * 16 + s
        _zero_slice(zeros16, accum, s)
        pltpu.sync_copy(eidx.at[wid], eidx_v)
        pltpu.sync_copy(ones_hbm, ones_v)
        plsc.subcore_barrier()

        @pl.loop(0, NB)
        def _(j):
            pltpu.async_copy(ones_v, accum.at[eidx_v.at[j, 1]], s0,
                             add=True).wait()

        plsc.subcore_barrier()
        _drain_slice(accum, out, c, s)

    return cntk


def _tc_a_body(x_ref, w_ref, b_ref, o_ref):
    o_ref[...] = jnp.maximum(_dot(x_ref[...], w_ref[...]) + b_ref[...], 0.0)


def _tc_a(x, w, b):
    return pl.pallas_call(
        _tc_a_body,
        grid=(GRID,),
        in_specs=[
            pl.BlockSpec((BLK, D), lambda i: (i, 0)),
            pl.BlockSpec((D, D), lambda i: (0, 0)),
            pl.BlockSpec((1, D), lambda i: (0, 0)),
        ],
        out_specs=pl.BlockSpec((BLK, D), lambda i: (i, 0)),
        out_shape=jax.ShapeDtypeStruct((N, D), jnp.float32),
    )(x, w, b)


def _tc_b_body(x_ref, p_ref, c_ref, awx_ref, awm_ref, ab_ref, lw_ref, lb_ref,
               h1_ref, t_ref, inv_ref):
    p = p_ref[0] + p_ref[1]
    cnt = (c_ref[0] + c_ref[1])[:, 0:1]
    inv = 1.0 / jnp.maximum(cnt, 1.0)
    mean = p * inv
    h = jnp.maximum(_dot(x_ref[...], awx_ref[...])
                    + _dot(mean, awm_ref[...]) + ab_ref[...], 0.0)
    nrm = jnp.sqrt(jnp.sum(h * h, axis=1, keepdims=True))
    h1 = h / jnp.maximum(nrm, 1e-12)
    h1_ref[...] = h1
    t_ref[...] = jnp.maximum(_dot(h1, lw_ref[...]) + lb_ref[...], 0.0)
    inv_ref[...] = jnp.broadcast_to(inv, (BLK, D))


def _tc_b(x, partials, cnt_partials, awx, awm, ab, lw, lb):
    return pl.pallas_call(
        _tc_b_body,
        grid=(GRID,),
        in_specs=[
            pl.BlockSpec((BLK, D), lambda i: (i, 0)),
            pl.BlockSpec((2, BLK, D), lambda i: (0, i, 0)),
            pl.BlockSpec((2, BLK, 16), lambda i: (0, i, 0)),
            pl.BlockSpec((D, D), lambda i: (0, 0)),
            pl.BlockSpec((D, D), lambda i: (0, 0)),
            pl.BlockSpec((1, D), lambda i: (0, 0)),
            pl.BlockSpec((D, D), lambda i: (0, 0)),
            pl.BlockSpec((1, D), lambda i: (0, 0)),
        ],
        out_specs=[
            pl.BlockSpec((BLK, D), lambda i: (i, 0)),
            pl.BlockSpec((BLK, D), lambda i: (i, 0)),
            pl.BlockSpec((BLK, D), lambda i: (i, 0)),
        ],
        out_shape=[
            jax.ShapeDtypeStruct((N, D), jnp.float32),
            jax.ShapeDtypeStruct((N, D), jnp.float32),
            jax.ShapeDtypeStruct((N, D), jnp.float32),
        ],
    )(x, partials, cnt_partials, awx, awm, ab, lw, lb)


def _tc_c_body(h1_ref, p_ref, inv_ref, awx_ref, awm_ref, ab_ref,
               pw1_ref, pb1_ref, pw2_ref, pb2_ref, o_ref):
    mean = (p_ref[0] + p_ref[1]) * inv_ref[...]
    h = jnp.maximum(_dot(h1_ref[...], awx_ref[...])
                    + _dot(mean, awm_ref[...]) + ab_ref[...], 0.0)
    nrm = jnp.sqrt(jnp.sum(h * h, axis=1, keepdims=True))
    h2 = h / jnp.maximum(nrm, 1e-12)
    h3 = _dot(h2, pw1_ref[...]) + pb1_ref[...]
    z = _dot(h3, pw2_ref[...]) + pb2_ref[...]
    m = jnp.max(z, axis=1, keepdims=True)
    lse = m + jnp.log(jnp.sum(jnp.exp(z - m), axis=1, keepdims=True))
    o_ref[...] = (z - lse)[:, :O_DIM]


def _tc_c(h1, partials, inv, awx, awm, ab, pw1, pb1, pw2, pb2):
    return pl.pallas_call(
        _tc_c_body,
        grid=(GRID,),
        in_specs=[
            pl.BlockSpec((BLK, D), lambda i: (i, 0)),
            pl.BlockSpec((2, BLK, D), lambda i: (0, i, 0)),
            pl.BlockSpec((BLK, D), lambda i: (i, 0)),
            pl.BlockSpec((D, D), lambda i: (0, 0)),
            pl.BlockSpec((D, D), lambda i: (0, 0)),
            pl.BlockSpec((1, D), lambda i: (0, 0)),
            pl.BlockSpec((D, D), lambda i: (0, 0)),
            pl.BlockSpec((1, D), lambda i: (0, 0)),
            pl.BlockSpec((D, D), lambda i: (0, 0)),
            pl.BlockSpec((1, D), lambda i: (0, 0)),
        ],
        out_specs=pl.BlockSpec((BLK, O_DIM), lambda i: (i, 0)),
        out_shape=jax.ShapeDtypeStruct((N, O_DIM), jnp.float32),
    )(h1, partials, inv, awx, awm, ab, pw1, pb1, pw2, pb2)


def kernel(x, edge_index, lin_W0, lin_b0, agg_W0, agg_b0,
           lin_W1, lin_b1, agg_W1, agg_b1,
           post_W1, post_b1, post_W2, post_b2):
    src = edge_index[0].astype(jnp.int32)
    dst = edge_index[1].astype(jnp.int32)
    pad = E_PAD - E
    src3 = jnp.concatenate(
        [src, jnp.arange(pad, dtype=jnp.int32) * 97 % N]
    ).reshape(NW, NB, EPB)
    dst3 = jnp.concatenate(
        [dst, N + (jnp.arange(pad, dtype=jnp.int32) % (NP - N))]
    ).reshape(NW, NB, EPB)
    eidx = jnp.stack([src3, dst3], axis=2)
    zeros128 = jnp.zeros((NP, D), jnp.float32)
    zeros16 = jnp.zeros((NP, 16), jnp.float32)
    ones16 = jnp.ones((EPB, 16), jnp.float32)

    lb0 = lin_b0.reshape(1, D)
    lb1 = lin_b1.reshape(1, D)
    ab0 = agg_b0.reshape(1, D)
    ab1 = agg_b1.reshape(1, D)
    pb1 = post_b1.reshape(1, D)
    pw2 = jnp.pad(post_W2, ((0, 0), (0, D - O_DIM)))
    pb2 = jnp.concatenate([post_b2,
                           jnp.full((D - O_DIM,), -1e30, jnp.float32)]
                          ).reshape(1, D)

    cntp = _make_edge_count()(eidx, zeros16, ones16)
    table0 = _tc_a(x, lin_W0, lb0)
    part0 = _make_seg_scatter()(table0, eidx, zeros128)
    h1, table1, inv = _tc_b(x, part0, cntp, agg_W0[:D], agg_W0[D:], ab0,
                            lin_W1, lb1)
    part1 = _make_seg_scatter()(table1, eidx, zeros128)
    return _tc_c(h1, part1, inv, agg_W1[:D], agg_W1[D:], ab1,
                 post_W1, pb1, pw2, pb2)


# speedup vs baseline: 7.2325x; 1.3197x over previous
import functools

import jax
import jax.numpy as jnp
from jax import lax
from jax.experimental import pallas as pl
from jax.experimental.pallas import tpu as pltpu
from jax.experimental.pallas import tpu_sc as plsc

N = 10000
D = 128
O_DIM = 40
E = 320000
NW = 32
EPB = 128
NB = 84
E_PAD = NW * NB * EPB
NBREAL = E // EPB
NP = 10000
RPT = 632
RPT_LAST = NP - 15 * RPT
BLK = 1000
GRID = N // BLK

_HIGH = jax.lax.Precision.HIGHEST


def _dot(a, b):
    return jnp.dot(a, b, precision=_HIGH, preferred_element_type=jnp.float32)


def _sc_mesh():
    return plsc.VectorSubcoreMesh(core_axis_name="c", subcore_axis_name="s",
                                  num_cores=2, num_subcores=16)


def _zero_slice(zeros, accum, s):
    @pl.when(s < 15)
    def _():
        pltpu.sync_copy(zeros.at[pl.ds(s * RPT, RPT)],
                        accum.at[pl.ds(s * RPT, RPT)])

    @pl.when(s == 15)
    def _():
        pltpu.sync_copy(zeros.at[pl.ds(15 * RPT, RPT_LAST)],
                        accum.at[pl.ds(15 * RPT, RPT_LAST)])


def _drain_slice(accum, out, c, s):
    @pl.when(s < 15)
    def _():
        pltpu.sync_copy(accum.at[pl.ds(s * RPT, RPT)],
                        out.at[c, pl.ds(s * RPT, RPT)])

    @pl.when(s == 15)
    def _():
        pltpu.sync_copy(accum.at[pl.ds(15 * RPT, RPT_LAST)],
                        out.at[c, pl.ds(15 * RPT, RPT_LAST)])


@functools.lru_cache(maxsize=None)
def _make_seg_scatter():
    @functools.partial(
        pl.kernel,
        out_type=jax.ShapeDtypeStruct((2, NP, D), jnp.float32),
        mesh=_sc_mesh(),
        scratch_types=[
            pltpu.VMEM((6, 2, EPB), jnp.int32),
            pltpu.VMEM((3, EPB, D), jnp.float32),
            pltpu.VMEM_SHARED((NP, D), jnp.float32),
            pltpu.SemaphoreType.DMA((6,)),
            pltpu.SemaphoreType.DMA((3,)),
            pltpu.SemaphoreType.DMA((3,)),
        ],
        compiler_params=pltpu.CompilerParams(use_tc_tiling_on_sc=False),
    )
    def seg(table, eidx, zeros, out, ib, rows, accum, isem, gsem, ssem):
        c = lax.axis_index("c")
        s = lax.axis_index("s")
        wid = c * 16 + s
        fb0 = wid * NB

        def i_issue(k, m):
            pltpu.async_copy(eidx.at[wid, k], ib.at[m], isem.at[m])

        def i_wait(k, m):
            pltpu.make_async_copy(eidx.at[wid, k], ib.at[m],
                                  isem.at[m]).wait()

        def g_issue(m, r):
            pltpu.async_copy(table.at[ib.at[m, 0]], rows.at[r], gsem.at[r])

        def g_wait(m, r):
            pltpu.make_async_copy(table.at[ib.at[m, 0]], rows.at[r],
                                  gsem.at[r]).wait()

        def s_issue(m, r):
            pltpu.async_copy(rows.at[r], accum.at[ib.at[m, 1]], ssem.at[r],
                             add=True)

        def s_wait(m, r):
            pltpu.make_async_copy(rows.at[r], accum.at[ib.at[m, 1]],
                                  ssem.at[r]).wait()

        for m in range(5):
            i_issue(m, m)
        for m in range(2):
            i_wait(m, m)
            g_issue(m, m)
        _zero_slice(zeros, accum, s)
        plsc.subcore_barrier()

        @pl.loop(0, NB, step=6)
        def _(j):
            for b in range(6):
                k = j + b
                g_wait(b, b % 3)

                @pl.when(fb0 + k < NBREAL)
                def _():
                    s_issue(b, b % 3)

                @pl.when((k >= 1) & (fb0 + k - 1 < NBREAL))
                def _():
                    s_wait((b - 1) % 6, (b - 1) % 3)

                @pl.when(k + 2 < NB)
                def _():
                    i_wait(k + 2, (b + 2) % 6)
                    g_issue((b + 2) % 6, (b + 2) % 3)

                @pl.when(k + 5 < NB)
                def _():
                    i_issue(k + 5, (b + 5) % 6)

        @pl.when(fb0 + NB - 1 < NBREAL)
        def _():
            s_wait((NB - 1) % 6, (NB - 1) % 3)

        plsc.subcore_barrier()
        _drain_slice(accum, out, c, s)

    return seg


@functools.lru_cache(maxsize=None)
def _make_edge_count():
    @functools.partial(
        pl.kernel,
        out_type=jax.ShapeDtypeStruct((2, NP, 16), jnp.float32),
        mesh=_sc_mesh(),
        scratch_types=[
            pltpu.VMEM((NB, 2, EPB), jnp.int32),
            pltpu.VMEM((EPB, 16), jnp.float32),
            pltpu.VMEM_SHARED((NP, 16), jnp.float32),
            pltpu.SemaphoreType.DMA,
        ],
        compiler_params=pltpu.CompilerParams(use_tc_tiling_on_sc=False),
    )
    def cntk(eidx, zeros16, ones_hbm, out, eidx_v, ones_v, accum, s0):
        c = lax.axis_index("c")
        s = lax.axis_index("s")
        wid = c * 16 + s
        fb0 = wid * NB
        _zero_slice(zeros16, accum, s)
        pltpu.sync_copy(eidx.at[wid], eidx_v)
        pltpu.sync_copy(ones_hbm, ones_v)
        plsc.subcore_barrier()

        @pl.loop(0, NB)
        def _(j):
            @pl.when(fb0 + j < NBREAL)
            def _():
                pltpu.async_copy(ones_v, accum.at[eidx_v.at[j, 1]], s0,
                                 add=True).wait()

        plsc.subcore_barrier()
        _drain_slice(accum, out, c, s)

    return cntk


def _tc_a_body(x_ref, w_ref, b_ref, o_ref):
    o_ref[...] = jnp.maximum(_dot(x_ref[...], w_ref[...]) + b_ref[...], 0.0)


def _tc_a(x, w, b):
    return pl.pallas_call(
        _tc_a_body,
        grid=(GRID,),
        in_specs=[
            pl.BlockSpec((BLK, D), lambda i: (i, 0)),
            pl.BlockSpec((D, D), lambda i: (0, 0)),
            pl.BlockSpec((1, D), lambda i: (0, 0)),
        ],
        out_specs=pl.BlockSpec((BLK, D), lambda i: (i, 0)),
        out_shape=jax.ShapeDtypeStruct((N, D), jnp.float32),
    )(x, w, b)


def _tc_b_body(x_ref, p_ref, c_ref, awx_ref, awm_ref, ab_ref, lw_ref, lb_ref,
               h1_ref, t_ref, inv_ref):
    p = p_ref[0] + p_ref[1]
    cnt = (c_ref[0] + c_ref[1])[:, 0:1]
    inv = 1.0 / jnp.maximum(cnt, 1.0)
    mean = p * inv
    h = jnp.maximum(_dot(x_ref[...], awx_ref[...])
                    + _dot(mean, awm_ref[...]) + ab_ref[...], 0.0)
    nrm = jnp.sqrt(jnp.sum(h * h, axis=1, keepdims=True))
    h1 = h / jnp.maximum(nrm, 1e-12)
    h1_ref[...] = h1
    t_ref[...] = jnp.maximum(_dot(h1, lw_ref[...]) + lb_ref[...], 0.0)
    inv_ref[...] = jnp.broadcast_to(inv, (BLK, D))


def _tc_b(x, partials, cnt_partials, awx, awm, ab, lw, lb):
    return pl.pallas_call(
        _tc_b_body,
        grid=(GRID,),
        in_specs=[
            pl.BlockSpec((BLK, D), lambda i: (i, 0)),
            pl.BlockSpec((2, BLK, D), lambda i: (0, i, 0)),
            pl.BlockSpec((2, BLK, 16), lambda i: (0, i, 0)),
            pl.BlockSpec((D, D), lambda i: (0, 0)),
            pl.BlockSpec((D, D), lambda i: (0, 0)),
            pl.BlockSpec((1, D), lambda i: (0, 0)),
            pl.BlockSpec((D, D), lambda i: (0, 0)),
            pl.BlockSpec((1, D), lambda i: (0, 0)),
        ],
        out_specs=[
            pl.BlockSpec((BLK, D), lambda i: (i, 0)),
            pl.BlockSpec((BLK, D), lambda i: (i, 0)),
            pl.BlockSpec((BLK, D), lambda i: (i, 0)),
        ],
        out_shape=[
            jax.ShapeDtypeStruct((N, D), jnp.float32),
            jax.ShapeDtypeStruct((N, D), jnp.float32),
            jax.ShapeDtypeStruct((N, D), jnp.float32),
        ],
    )(x, partials, cnt_partials, awx, awm, ab, lw, lb)


def _tc_c_body(h1_ref, p_ref, inv_ref, awx_ref, awm_ref, ab_ref,
               pw1_ref, pb1_ref, pw2_ref, pb2_ref, o_ref):
    mean = (p_ref[0] + p_ref[1]) * inv_ref[...]
    h = jnp.maximum(_dot(h1_ref[...], awx_ref[...])
                    + _dot(mean, awm_ref[...]) + ab_ref[...], 0.0)
    nrm = jnp.sqrt(jnp.sum(h * h, axis=1, keepdims=True))
    h2 = h / jnp.maximum(nrm, 1e-12)
    h3 = _dot(h2, pw1_ref[...]) + pb1_ref[...]
    z = _dot(h3, pw2_ref[...]) + pb2_ref[...]
    m = jnp.max(z, axis=1, keepdims=True)
    lse = m + jnp.log(jnp.sum(jnp.exp(z - m), axis=1, keepdims=True))
    o_ref[...] = (z - lse)[:, :O_DIM]


def _tc_c(h1, partials, inv, awx, awm, ab, pw1, pb1, pw2, pb2):
    return pl.pallas_call(
        _tc_c_body,
        grid=(GRID,),
        in_specs=[
            pl.BlockSpec((BLK, D), lambda i: (i, 0)),
            pl.BlockSpec((2, BLK, D), lambda i: (0, i, 0)),
            pl.BlockSpec((BLK, D), lambda i: (i, 0)),
            pl.BlockSpec((D, D), lambda i: (0, 0)),
            pl.BlockSpec((D, D), lambda i: (0, 0)),
            pl.BlockSpec((1, D), lambda i: (0, 0)),
            pl.BlockSpec((D, D), lambda i: (0, 0)),
            pl.BlockSpec((1, D), lambda i: (0, 0)),
            pl.BlockSpec((D, D), lambda i: (0, 0)),
            pl.BlockSpec((1, D), lambda i: (0, 0)),
        ],
        out_specs=pl.BlockSpec((BLK, O_DIM), lambda i: (i, 0)),
        out_shape=jax.ShapeDtypeStruct((N, O_DIM), jnp.float32),
    )(h1, partials, inv, awx, awm, ab, pw1, pb1, pw2, pb2)


def kernel(x, edge_index, lin_W0, lin_b0, agg_W0, agg_b0,
           lin_W1, lin_b1, agg_W1, agg_b1,
           post_W1, post_b1, post_W2, post_b2):
    src = edge_index[0].astype(jnp.int32)
    dst = edge_index[1].astype(jnp.int32)
    pad = E_PAD - E
    fill = jnp.arange(pad, dtype=jnp.int32) * 97 % N
    src3 = jnp.concatenate([src, fill]).reshape(NW, NB, EPB)
    dst3 = jnp.concatenate([dst, fill]).reshape(NW, NB, EPB)
    eidx = jnp.stack([src3, dst3], axis=2)
    zeros128 = jnp.zeros((NP, D), jnp.float32)
    zeros16 = jnp.zeros((NP, 16), jnp.float32)
    ones16 = jnp.ones((EPB, 16), jnp.float32)

    lb0 = lin_b0.reshape(1, D)
    lb1 = lin_b1.reshape(1, D)
    ab0 = agg_b0.reshape(1, D)
    ab1 = agg_b1.reshape(1, D)
    pb1 = post_b1.reshape(1, D)
    pw2 = jnp.pad(post_W2, ((0, 0), (0, D - O_DIM)))
    pb2 = jnp.concatenate([post_b2,
                           jnp.full((D - O_DIM,), -1e30, jnp.float32)]
                          ).reshape(1, D)

    cntp = _make_edge_count()(eidx, zeros16, ones16)
    table0 = _tc_a(x, lin_W0, lb0)
    part0 = _make_seg_scatter()(table0, eidx, zeros128)
    h1, table1, inv = _tc_b(x, part0, cntp, agg_W0[:D], agg_W0[D:], ab0,
                            lin_W1, lb1)
    part1 = _make_seg_scatter()(table1, eidx, zeros128)
    return _tc_c(h1, part1, inv, agg_W1[:D], agg_W1[D:], ab1,
                 post_W1, pb1, pw2, pb2)


# speedup vs baseline: 7.6808x; 1.0620x over previous
import functools

import jax
import jax.numpy as jnp
from jax import lax
from jax.experimental import pallas as pl
from jax.experimental.pallas import tpu as pltpu
from jax.experimental.pallas import tpu_sc as plsc

N = 10000
D = 128
O_DIM = 40
E = 320000
NW = 32
EPB = 128
NBREAL = E // EPB
AW = -(-NBREAL // NW)
NB = 84
NP = 10000
RPT = 632
RPT_LAST = NP - 15 * RPT
BLK = 1000
GRID = N // BLK

_HIGH = jax.lax.Precision.HIGHEST


def _dot(a, b):
    return jnp.dot(a, b, precision=_HIGH, preferred_element_type=jnp.float32)


def _sc_mesh():
    return plsc.VectorSubcoreMesh(core_axis_name="c", subcore_axis_name="s",
                                  num_cores=2, num_subcores=16)


def _zero_slice(zeros, accum, s):
    @pl.when(s < 15)
    def _():
        pltpu.sync_copy(zeros.at[pl.ds(s * RPT, RPT)],
                        accum.at[pl.ds(s * RPT, RPT)])

    @pl.when(s == 15)
    def _():
        pltpu.sync_copy(zeros.at[pl.ds(15 * RPT, RPT_LAST)],
                        accum.at[pl.ds(15 * RPT, RPT_LAST)])


def _drain_slice(accum, out, c, s):
    @pl.when(s < 15)
    def _():
        pltpu.sync_copy(accum.at[pl.ds(s * RPT, RPT)],
                        out.at[c, pl.ds(s * RPT, RPT)])

    @pl.when(s == 15)
    def _():
        pltpu.sync_copy(accum.at[pl.ds(15 * RPT, RPT_LAST)],
                        out.at[c, pl.ds(15 * RPT, RPT_LAST)])


@functools.lru_cache(maxsize=None)
def _make_seg_scatter():
    @functools.partial(
        pl.kernel,
        out_type=jax.ShapeDtypeStruct((2, NP, D), jnp.float32),
        mesh=_sc_mesh(),
        scratch_types=[
            pltpu.VMEM((6, 2, EPB), jnp.int32),
            pltpu.VMEM((3, EPB, D), jnp.float32),
            pltpu.VMEM_SHARED((NP, D), jnp.float32),
            pltpu.SemaphoreType.DMA((6,)),
            pltpu.SemaphoreType.DMA((6,)),
            pltpu.SemaphoreType.DMA((3,)),
            pltpu.SemaphoreType.DMA((3,)),
        ],
        compiler_params=pltpu.CompilerParams(use_tc_tiling_on_sc=False),
    )
    def seg(table, eidx, zeros, out, ib, rows, accum, isems, isemd, gsem,
            ssem):
        c = lax.axis_index("c")
        s = lax.axis_index("s")
        wid = c * 16 + s
        fb0 = wid * AW
        lim = jnp.minimum(AW, NBREAL - fb0)

        def i_issue(k, m):
            pltpu.async_copy(eidx.at[0, fb0 + k], ib.at[m, 0], isems.at[m])
            pltpu.async_copy(eidx.at[1, fb0 + k], ib.at[m, 1], isemd.at[m])

        def i_wait(k, m):
            pltpu.make_async_copy(eidx.at[0, fb0 + k], ib.at[m, 0],
                                  isems.at[m]).wait()
            pltpu.make_async_copy(eidx.at[1, fb0 + k], ib.at[m, 1],
                                  isemd.at[m]).wait()

        def g_issue(m, r):
            pltpu.async_copy(table.at[ib.at[m, 0]], rows.at[r], gsem.at[r])

        def g_wait(m, r):
            pltpu.make_async_copy(table.at[ib.at[m, 0]], rows.at[r],
                                  gsem.at[r]).wait()

        def s_issue(m, r):
            pltpu.async_copy(rows.at[r], accum.at[ib.at[m, 1]], ssem.at[r],
                             add=True)

        def s_wait(m, r):
            pltpu.make_async_copy(rows.at[r], accum.at[ib.at[m, 1]],
                                  ssem.at[r]).wait()

        for m in range(5):
            @pl.when(m < lim)
            def _():
                i_issue(m, m)
        for m in range(2):
            @pl.when(m < lim)
            def _():
                i_wait(m, m)
                g_issue(m, m)
        _zero_slice(zeros, accum, s)
        plsc.subcore_barrier()

        @pl.loop(0, NB, step=6)
        def _(j):
            for b in range(6):
                k = j + b

                @pl.when(k < lim)
                def _():
                    g_wait(b, b % 3)
                    s_issue(b, b % 3)

                @pl.when((k >= 1) & (k - 1 < lim))
                def _():
                    s_wait((b - 1) % 6, (b - 1) % 3)

                @pl.when(k + 2 < lim)
                def _():
                    i_wait(k + 2, (b + 2) % 6)
                    g_issue((b + 2) % 6, (b + 2) % 3)

                @pl.when(k + 5 < lim)
                def _():
                    i_issue(k + 5, (b + 5) % 6)

        plsc.subcore_barrier()
        _drain_slice(accum, out, c, s)

    return seg


@functools.lru_cache(maxsize=None)
def _make_edge_count():
    @functools.partial(
        pl.kernel,
        out_type=jax.ShapeDtypeStruct((2, NP, 16), jnp.float32),
        mesh=_sc_mesh(),
        scratch_types=[
            pltpu.VMEM((AW, EPB), jnp.int32),
            pltpu.VMEM((EPB, 16), jnp.float32),
            pltpu.VMEM_SHARED((NP, 16), jnp.float32),
            pltpu.SemaphoreType.DMA,
        ],
        compiler_params=pltpu.CompilerParams(use_tc_tiling_on_sc=False),
    )
    def cntk(eidxp, zeros16, ones_hbm, out, dst_v, ones_v, accum, s0):
        c = lax.axis_index("c")
        s = lax.axis_index("s")
        wid = c * 16 + s
        fb0 = wid * AW
        lim = jnp.minimum(AW, NBREAL - fb0)
        _zero_slice(zeros16, accum, s)
        pltpu.sync_copy(eidxp.at[1, pl.ds(fb0, AW)], dst_v)
        pltpu.sync_copy(ones_hbm, ones_v)
        plsc.subcore_barrier()

        @pl.loop(0, AW)
        def _(j):
            @pl.when(j < lim)
            def _():
                pltpu.async_copy(ones_v, accum.at[dst_v.at[j]], s0,
                                 add=True).wait()

        plsc.subcore_barrier()
        _drain_slice(accum, out, c, s)

    return cntk


def _tc_a_body(x_ref, w_ref, b_ref, o_ref):
    o_ref[...] = jnp.maximum(_dot(x_ref[...], w_ref[...]) + b_ref[...], 0.0)


def _tc_a(x, w, b):
    return pl.pallas_call(
        _tc_a_body,
        grid=(GRID,),
        in_specs=[
            pl.BlockSpec((BLK, D), lambda i: (i, 0)),
            pl.BlockSpec((D, D), lambda i: (0, 0)),
            pl.BlockSpec((1, D), lambda i: (0, 0)),
        ],
        out_specs=pl.BlockSpec((BLK, D), lambda i: (i, 0)),
        out_shape=jax.ShapeDtypeStruct((N, D), jnp.float32),
    )(x, w, b)


def _tc_b_body(x_ref, p_ref, c_ref, awx_ref, awm_ref, ab_ref, lw_ref, lb_ref,
               h1_ref, t_ref, inv_ref):
    p = p_ref[0] + p_ref[1]
    inv = 1.0 / jnp.maximum(c_ref[...], 1.0)
    mean = p * inv
    h = jnp.maximum(_dot(x_ref[...], awx_ref[...])
                    + _dot(mean, awm_ref[...]) + ab_ref[...], 0.0)
    nrm = jnp.sqrt(jnp.sum(h * h, axis=1, keepdims=True))
    h1 = h / jnp.maximum(nrm, 1e-12)
    h1_ref[...] = h1
    t_ref[...] = jnp.maximum(_dot(h1, lw_ref[...]) + lb_ref[...], 0.0)
    inv_ref[...] = inv


def _tc_b(x, partials, cnt_partials, awx, awm, ab, lw, lb):
    return pl.pallas_call(
        _tc_b_body,
        grid=(GRID,),
        in_specs=[
            pl.BlockSpec((BLK, D), lambda i: (i, 0)),
            pl.BlockSpec((2, BLK, D), lambda i: (0, i, 0)),
            pl.BlockSpec((BLK, 1), lambda i: (i, 0)),
            pl.BlockSpec((D, D), lambda i: (0, 0)),
            pl.BlockSpec((D, D), lambda i: (0, 0)),
            pl.BlockSpec((1, D), lambda i: (0, 0)),
            pl.BlockSpec((D, D), lambda i: (0, 0)),
            pl.BlockSpec((1, D), lambda i: (0, 0)),
        ],
        out_specs=[
            pl.BlockSpec((BLK, D), lambda i: (i, 0)),
            pl.BlockSpec((BLK, D), lambda i: (i, 0)),
            pl.BlockSpec((BLK, 1), lambda i: (i, 0)),
        ],
        out_shape=[
            jax.ShapeDtypeStruct((N, D), jnp.float32),
            jax.ShapeDtypeStruct((N, D), jnp.float32),
            jax.ShapeDtypeStruct((N, 1), jnp.float32),
        ],
    )(x, partials, cnt_partials, awx, awm, ab, lw, lb)


def _tc_c_body(h1_ref, p_ref, inv_ref, awx_ref, awm_ref, ab_ref,
               pw1_ref, pb1_ref, pw2_ref, pb2_ref, o_ref):
    mean = (p_ref[0] + p_ref[1]) * inv_ref[...]
    h = jnp.maximum(_dot(h1_ref[...], awx_ref[...])
                    + _dot(mean, awm_ref[...]) + ab_ref[...], 0.0)
    nrm = jnp.sqrt(jnp.sum(h * h, axis=1, keepdims=True))
    h2 = h / jnp.maximum(nrm, 1e-12)
    h3 = _dot(h2, pw1_ref[...]) + pb1_ref[...]
    z = _dot(h3, pw2_ref[...]) + pb2_ref[...]
    m = jnp.max(z, axis=1, keepdims=True)
    lse = m + jnp.log(jnp.sum(jnp.exp(z - m), axis=1, keepdims=True))
    o_ref[...] = (z - lse)[:, :O_DIM]


def _tc_c(h1, partials, inv, awx, awm, ab, pw1, pb1, pw2, pb2):
    return pl.pallas_call(
        _tc_c_body,
        grid=(GRID,),
        in_specs=[
            pl.BlockSpec((BLK, D), lambda i: (i, 0)),
            pl.BlockSpec((2, BLK, D), lambda i: (0, i, 0)),
            pl.BlockSpec((BLK, 1), lambda i: (i, 0)),
            pl.BlockSpec((D, D), lambda i: (0, 0)),
            pl.BlockSpec((D, D), lambda i: (0, 0)),
            pl.BlockSpec((1, D), lambda i: (0, 0)),
            pl.BlockSpec((D, D), lambda i: (0, 0)),
            pl.BlockSpec((1, D), lambda i: (0, 0)),
            pl.BlockSpec((D, D), lambda i: (0, 0)),
            pl.BlockSpec((1, D), lambda i: (0, 0)),
        ],
        out_specs=pl.BlockSpec((BLK, O_DIM), lambda i: (i, 0)),
        out_shape=jax.ShapeDtypeStruct((N, O_DIM), jnp.float32),
    )(h1, partials, inv, awx, awm, ab, pw1, pb1, pw2, pb2)


def kernel(x, edge_index, lin_W0, lin_b0, agg_W0, agg_b0,
           lin_W1, lin_b1, agg_W1, agg_b1,
           post_W1, post_b1, post_W2, post_b2):
    eidx = edge_index.astype(jnp.int32).reshape(2, NBREAL, EPB)
    eidxp = jnp.pad(eidx, ((0, 0), (0, NW * AW - NBREAL), (0, 0)))
    zeros128 = jnp.zeros((NP, D), jnp.float32)
    zeros16 = jnp.zeros((NP, 16), jnp.float32)
    ones16 = jnp.ones((EPB, 16), jnp.float32)

    lb0 = lin_b0.reshape(1, D)
    lb1 = lin_b1.reshape(1, D)
    ab0 = agg_b0.reshape(1, D)
    ab1 = agg_b1.reshape(1, D)
    pb1 = post_b1.reshape(1, D)
    pw2 = jnp.pad(post_W2, ((0, 0), (0, D - O_DIM)))
    pb2 = jnp.concatenate([post_b2,
                           jnp.full((D - O_DIM,), -1e30, jnp.float32)]
                          ).reshape(1, D)

    cntp = _make_edge_count()(eidxp, zeros16, ones16)
    cnt_col = cntp[0, :, 0:1] + cntp[1, :, 0:1]
    table0 = _tc_a(x, lin_W0, lb0)
    part0 = _make_seg_scatter()(table0, eidx, zeros128)
    h1, table1, inv = _tc_b(x, part0, cnt_col, agg_W0[:D], agg_W0[D:], ab0,
                            lin_W1, lb1)
    part1 = _make_seg_scatter()(table1, eidx, zeros128)
    return _tc_c(h1, part1, inv, agg_W1[:D], agg_W1[D:], ab1,
                 post_W1, pb1, pw2, pb2)


# speedup vs baseline: 9.4721x; 1.2332x over previous
import functools

import jax
import jax.numpy as jnp
from jax import lax
from jax.experimental import pallas as pl
from jax.experimental.pallas import tpu as pltpu
from jax.experimental.pallas import tpu_sc as plsc

N = 10000
D = 128
O_DIM = 40
E = 320000
NW = 32
EPB = 128
NBREAL = E // EPB
AW = -(-NBREAL // NW)
NB = 84
NP = 10000
RPT = 632
RPT_LAST = NP - 15 * RPT
BLK = 1000
GRID = N // BLK

def _dot(a, b):
    return jnp.dot(a, b, preferred_element_type=jnp.float32)


def _sc_mesh():
    return plsc.VectorSubcoreMesh(core_axis_name="c", subcore_axis_name="s",
                                  num_cores=2, num_subcores=16)


def _zero_slice(zeros, accum, s):
    @pl.when(s < 15)
    def _():
        pltpu.sync_copy(zeros.at[pl.ds(s * RPT, RPT)],
                        accum.at[pl.ds(s * RPT, RPT)])

    @pl.when(s == 15)
    def _():
        pltpu.sync_copy(zeros.at[pl.ds(15 * RPT, RPT_LAST)],
                        accum.at[pl.ds(15 * RPT, RPT_LAST)])


def _drain_slice(accum, out, c, s):
    @pl.when(s < 15)
    def _():
        pltpu.sync_copy(accum.at[pl.ds(s * RPT, RPT)],
                        out.at[c, pl.ds(s * RPT, RPT)])

    @pl.when(s == 15)
    def _():
        pltpu.sync_copy(accum.at[pl.ds(15 * RPT, RPT_LAST)],
                        out.at[c, pl.ds(15 * RPT, RPT_LAST)])


@functools.lru_cache(maxsize=None)
def _make_seg_scatter():
    @functools.partial(
        pl.kernel,
        out_type=jax.ShapeDtypeStruct((2, NP, D), jnp.float32),
        mesh=_sc_mesh(),
        scratch_types=[
            pltpu.VMEM((6, 2, EPB), jnp.int32),
            pltpu.VMEM((3, EPB, D), jnp.float32),
            pltpu.VMEM_SHARED((NP, D), jnp.float32),
            pltpu.SemaphoreType.DMA((6,)),
            pltpu.SemaphoreType.DMA((6,)),
            pltpu.SemaphoreType.DMA((3,)),
            pltpu.SemaphoreType.DMA((3,)),
        ],
        compiler_params=pltpu.CompilerParams(use_tc_tiling_on_sc=False),
    )
    def seg(table, eidx, zeros, out, ib, rows, accum, isems, isemd, gsem,
            ssem):
        c = lax.axis_index("c")
        s = lax.axis_index("s")
        wid = c * 16 + s
        fb0 = wid * AW
        lim = jnp.minimum(AW, NBREAL - fb0)

        def i_issue(k, m):
            pltpu.async_copy(eidx.at[0, fb0 + k], ib.at[m, 0], isems.at[m])
            pltpu.async_copy(eidx.at[1, fb0 + k], ib.at[m, 1], isemd.at[m])

        def i_wait(k, m):
            pltpu.make_async_copy(eidx.at[0, fb0 + k], ib.at[m, 0],
                                  isems.at[m]).wait()
            pltpu.make_async_copy(eidx.at[1, fb0 + k], ib.at[m, 1],
                                  isemd.at[m]).wait()

        def g_issue(m, r):
            pltpu.async_copy(table.at[ib.at[m, 0]], rows.at[r], gsem.at[r])

        def g_wait(m, r):
            pltpu.make_async_copy(table.at[ib.at[m, 0]], rows.at[r],
                                  gsem.at[r]).wait()

        def s_issue(m, r):
            pltpu.async_copy(rows.at[r], accum.at[ib.at[m, 1]], ssem.at[r],
                             add=True)

        def s_wait(m, r):
            pltpu.make_async_copy(rows.at[r], accum.at[ib.at[m, 1]],
                                  ssem.at[r]).wait()

        for m in range(5):
            @pl.when(m < lim)
            def _():
                i_issue(m, m)
        for m in range(2):
            @pl.when(m < lim)
            def _():
                i_wait(m, m)
                g_issue(m, m)
        _zero_slice(zeros, accum, s)
        plsc.subcore_barrier()

        @pl.loop(0, NB, step=6)
        def _(j):
            for b in range(6):
                k = j + b

                @pl.when(k < lim)
                def _():
                    g_wait(b, b % 3)
                    s_issue(b, b % 3)

                @pl.when((k >= 1) & (k - 1 < lim))
                def _():
                    s_wait((b - 1) % 6, (b - 1) % 3)

                @pl.when(k + 2 < lim)
                def _():
                    i_wait(k + 2, (b + 2) % 6)
                    g_issue((b + 2) % 6, (b + 2) % 3)

                @pl.when(k + 5 < lim)
                def _():
                    i_issue(k + 5, (b + 5) % 6)

        plsc.subcore_barrier()
        _drain_slice(accum, out, c, s)

    return seg


@functools.lru_cache(maxsize=None)
def _make_edge_count():
    @functools.partial(
        pl.kernel,
        out_type=jax.ShapeDtypeStruct((2, NP, 16), jnp.float32),
        mesh=_sc_mesh(),
        scratch_types=[
            pltpu.VMEM((AW, EPB), jnp.int32),
            pltpu.VMEM((EPB, 16), jnp.float32),
            pltpu.VMEM_SHARED((NP, 16), jnp.float32),
            pltpu.SemaphoreType.DMA,
        ],
        compiler_params=pltpu.CompilerParams(use_tc_tiling_on_sc=False),
    )
    def cntk(eidxp, zeros16, ones_hbm, out, dst_v, ones_v, accum, s0):
        c = lax.axis_index("c")
        s = lax.axis_index("s")
        wid = c * 16 + s
        fb0 = wid * AW
        lim = jnp.minimum(AW, NBREAL - fb0)
        _zero_slice(zeros16, accum, s)
        pltpu.sync_copy(eidxp.at[1, pl.ds(fb0, AW)], dst_v)
        pltpu.sync_copy(ones_hbm, ones_v)
        plsc.subcore_barrier()

        @pl.loop(0, AW)
        def _(j):
            @pl.when(j < lim)
            def _():
                pltpu.async_copy(ones_v, accum.at[dst_v.at[j]], s0,
                                 add=True).wait()

        plsc.subcore_barrier()
        _drain_slice(accum, out, c, s)

    return cntk


def _tc_a_body(x_ref, w_ref, b_ref, o_ref):
    o_ref[...] = jnp.maximum(_dot(x_ref[...], w_ref[...]) + b_ref[...], 0.0)


def _tc_a(x, w, b):
    return pl.pallas_call(
        _tc_a_body,
        grid=(GRID,),
        in_specs=[
            pl.BlockSpec((BLK, D), lambda i: (i, 0)),
            pl.BlockSpec((D, D), lambda i: (0, 0)),
            pl.BlockSpec((1, D), lambda i: (0, 0)),
        ],
        out_specs=pl.BlockSpec((BLK, D), lambda i: (i, 0)),
        out_shape=jax.ShapeDtypeStruct((N, D), jnp.float32),
    )(x, w, b)


def _tc_b_body(x_ref, p_ref, c_ref, awx_ref, awm_ref, ab_ref, lw_ref, lb_ref,
               h1_ref, t_ref, inv_ref):
    p = p_ref[0] + p_ref[1]
    inv = 1.0 / jnp.maximum(c_ref[...], 1.0)
    mean = p * inv
    h = jnp.maximum(_dot(x_ref[...], awx_ref[...])
                    + _dot(mean, awm_ref[...]) + ab_ref[...], 0.0)
    nrm = jnp.sqrt(jnp.sum(h * h, axis=1, keepdims=True))
    h1 = h / jnp.maximum(nrm, 1e-12)
    h1_ref[...] = h1
    t_ref[...] = jnp.maximum(_dot(h1, lw_ref[...]) + lb_ref[...], 0.0)
    inv_ref[...] = inv


def _tc_b(x, partials, cnt_partials, awx, awm, ab, lw, lb):
    return pl.pallas_call(
        _tc_b_body,
        grid=(GRID,),
        in_specs=[
            pl.BlockSpec((BLK, D), lambda i: (i, 0)),
            pl.BlockSpec((2, BLK, D), lambda i: (0, i, 0)),
            pl.BlockSpec((BLK, 1), lambda i: (i, 0)),
            pl.BlockSpec((D, D), lambda i: (0, 0)),
            pl.BlockSpec((D, D), lambda i: (0, 0)),
            pl.BlockSpec((1, D), lambda i: (0, 0)),
            pl.BlockSpec((D, D), lambda i: (0, 0)),
            pl.BlockSpec((1, D), lambda i: (0, 0)),
        ],
        out_specs=[
            pl.BlockSpec((BLK, D), lambda i: (i, 0)),
            pl.BlockSpec((BLK, D), lambda i: (i, 0)),
            pl.BlockSpec((BLK, 1), lambda i: (i, 0)),
        ],
        out_shape=[
            jax.ShapeDtypeStruct((N, D), jnp.float32),
            jax.ShapeDtypeStruct((N, D), jnp.float32),
            jax.ShapeDtypeStruct((N, 1), jnp.float32),
        ],
    )(x, partials, cnt_partials, awx, awm, ab, lw, lb)


def _tc_c_body(h1_ref, p_ref, inv_ref, awx_ref, awm_ref, ab_ref,
               pw1_ref, pb1_ref, pw2_ref, pb2_ref, o_ref):
    mean = (p_ref[0] + p_ref[1]) * inv_ref[...]
    h = jnp.maximum(_dot(h1_ref[...], awx_ref[...])
                    + _dot(mean, awm_ref[...]) + ab_ref[...], 0.0)
    nrm = jnp.sqrt(jnp.sum(h * h, axis=1, keepdims=True))
    h2 = h / jnp.maximum(nrm, 1e-12)
    h3 = _dot(h2, pw1_ref[...]) + pb1_ref[...]
    z = _dot(h3, pw2_ref[...]) + pb2_ref[...]
    m = jnp.max(z, axis=1, keepdims=True)
    lse = m + jnp.log(jnp.sum(jnp.exp(z - m), axis=1, keepdims=True))
    o_ref[...] = (z - lse)[:, :O_DIM]


def _tc_c(h1, partials, inv, awx, awm, ab, pw1, pb1, pw2, pb2):
    return pl.pallas_call(
        _tc_c_body,
        grid=(GRID,),
        in_specs=[
            pl.BlockSpec((BLK, D), lambda i: (i, 0)),
            pl.BlockSpec((2, BLK, D), lambda i: (0, i, 0)),
            pl.BlockSpec((BLK, 1), lambda i: (i, 0)),
            pl.BlockSpec((D, D), lambda i: (0, 0)),
            pl.BlockSpec((D, D), lambda i: (0, 0)),
            pl.BlockSpec((1, D), lambda i: (0, 0)),
            pl.BlockSpec((D, D), lambda i: (0, 0)),
            pl.BlockSpec((1, D), lambda i: (0, 0)),
            pl.BlockSpec((D, D), lambda i: (0, 0)),
            pl.BlockSpec((1, D), lambda i: (0, 0)),
        ],
        out_specs=pl.BlockSpec((BLK, O_DIM), lambda i: (i, 0)),
        out_shape=jax.ShapeDtypeStruct((N, O_DIM), jnp.float32),
    )(h1, partials, inv, awx, awm, ab, pw1, pb1, pw2, pb2)


def kernel(x, edge_index, lin_W0, lin_b0, agg_W0, agg_b0,
           lin_W1, lin_b1, agg_W1, agg_b1,
           post_W1, post_b1, post_W2, post_b2):
    eidx = edge_index.astype(jnp.int32).reshape(2, NBREAL, EPB)
    eidxp = jnp.pad(eidx, ((0, 0), (0, NW * AW - NBREAL), (0, 0)))
    zeros128 = jnp.zeros((NP, D), jnp.float32)
    zeros16 = jnp.zeros((NP, 16), jnp.float32)
    ones16 = jnp.ones((EPB, 16), jnp.float32)

    lb0 = lin_b0.reshape(1, D)
    lb1 = lin_b1.reshape(1, D)
    ab0 = agg_b0.reshape(1, D)
    ab1 = agg_b1.reshape(1, D)
    pb1 = post_b1.reshape(1, D)
    pw2 = jnp.pad(post_W2, ((0, 0), (0, D - O_DIM)))
    pb2 = jnp.concatenate([post_b2,
                           jnp.full((D - O_DIM,), -1e30, jnp.float32)]
                          ).reshape(1, D)

    cntp = _make_edge_count()(eidxp, zeros16, ones16)
    cnt_col = cntp[0, :, 0:1] + cntp[1, :, 0:1]
    table0 = _tc_a(x, lin_W0, lb0)
    part0 = _make_seg_scatter()(table0, eidx, zeros128)
    h1, table1, inv = _tc_b(x, part0, cnt_col, agg_W0[:D], agg_W0[D:], ab0,
                            lin_W1, lb1)
    part1 = _make_seg_scatter()(table1, eidx, zeros128)
    return _tc_c(h1, part1, inv, agg_W1[:D], agg_W1[D:], ab1,
                 post_W1, pb1, pw2, pb2)


# speedup vs baseline: 9.5880x; 1.0122x over previous
import functools

import jax
import jax.numpy as jnp
from jax import lax
from jax.experimental import pallas as pl
from jax.experimental.pallas import tpu as pltpu
from jax.experimental.pallas import tpu_sc as plsc

N = 10000
D = 128
O_DIM = 40
E = 320000
NW = 32
EPB = 128
NBREAL = E // EPB
AW = -(-NBREAL // NW)
NB = 84
NP = 10000
RPT = 632
RPT_LAST = NP - 15 * RPT
BLK = 1000
GRID = N // BLK

def _dot(a, b):
    return jnp.dot(a, b, preferred_element_type=jnp.float32)


def _sc_mesh():
    return plsc.VectorSubcoreMesh(core_axis_name="c", subcore_axis_name="s",
                                  num_cores=2, num_subcores=16)


def _zero_slice(zeros, accum, s):
    @pl.when(s < 15)
    def _():
        pltpu.sync_copy(zeros.at[pl.ds(s * RPT, RPT)],
                        accum.at[pl.ds(s * RPT, RPT)])

    @pl.when(s == 15)
    def _():
        pltpu.sync_copy(zeros.at[pl.ds(15 * RPT, RPT_LAST)],
                        accum.at[pl.ds(15 * RPT, RPT_LAST)])


def _drain_slice(accum, out, c, s):
    @pl.when(s < 15)
    def _():
        pltpu.sync_copy(accum.at[pl.ds(s * RPT, RPT)],
                        out.at[c, pl.ds(s * RPT, RPT)])

    @pl.when(s == 15)
    def _():
        pltpu.sync_copy(accum.at[pl.ds(15 * RPT, RPT_LAST)],
                        out.at[c, pl.ds(15 * RPT, RPT_LAST)])


@functools.lru_cache(maxsize=None)
def _make_seg_scatter():
    @functools.partial(
        pl.kernel,
        out_type=jax.ShapeDtypeStruct((2, NP, D), jnp.float32),
        mesh=_sc_mesh(),
        scratch_types=[
            pltpu.VMEM((6, 2, EPB), jnp.int32),
            pltpu.VMEM((3, EPB, D), jnp.float32),
            pltpu.VMEM_SHARED((NP, D), jnp.float32),
            pltpu.SemaphoreType.DMA((6,)),
            pltpu.SemaphoreType.DMA((6,)),
            pltpu.SemaphoreType.DMA((3,)),
            pltpu.SemaphoreType.DMA((3,)),
        ],
        compiler_params=pltpu.CompilerParams(use_tc_tiling_on_sc=False),
    )
    def seg(table, eidx, zeros, out, ib, rows, accum, isems, isemd, gsem,
            ssem):
        c = lax.axis_index("c")
        s = lax.axis_index("s")
        wid = c * 16 + s
        fb0 = wid * AW
        lim = jnp.minimum(AW, NBREAL - fb0)

        def i_issue(k, m):
            pltpu.async_copy(eidx.at[0, fb0 + k], ib.at[m, 0], isems.at[m])
            pltpu.async_copy(eidx.at[1, fb0 + k], ib.at[m, 1], isemd.at[m])

        def i_wait(k, m):
            pltpu.make_async_copy(eidx.at[0, fb0 + k], ib.at[m, 0],
                                  isems.at[m]).wait()
            pltpu.make_async_copy(eidx.at[1, fb0 + k], ib.at[m, 1],
                                  isemd.at[m]).wait()

        def g_issue(m, r):
            pltpu.async_copy(table.at[ib.at[m, 0]], rows.at[r], gsem.at[r])

        def g_wait(m, r):
            pltpu.make_async_copy(table.at[ib.at[m, 0]], rows.at[r],
                                  gsem.at[r]).wait()

        def s_issue(m, r):
            pltpu.async_copy(rows.at[r], accum.at[ib.at[m, 1]], ssem.at[r],
                             add=True)

        def s_wait(m, r):
            pltpu.make_async_copy(rows.at[r], accum.at[ib.at[m, 1]],
                                  ssem.at[r]).wait()

        for m in range(5):
            @pl.when(m < lim)
            def _():
                i_issue(m, m)
        for m in range(2):
            @pl.when(m < lim)
            def _():
                i_wait(m, m)
                g_issue(m, m)
        _zero_slice(zeros, accum, s)
        plsc.subcore_barrier()

        @pl.loop(0, NB, step=6)
        def _(j):
            for b in range(6):
                k = j + b

                @pl.when(k < lim)
                def _():
                    g_wait(b, b % 3)
                    s_issue(b, b % 3)

                @pl.when((k >= 1) & (k - 1 < lim))
                def _():
                    s_wait((b - 1) % 6, (b - 1) % 3)

                @pl.when(k + 2 < lim)
                def _():
                    i_wait(k + 2, (b + 2) % 6)
                    g_issue((b + 2) % 6, (b + 2) % 3)

                @pl.when(k + 5 < lim)
                def _():
                    i_issue(k + 5, (b + 5) % 6)

        plsc.subcore_barrier()
        _drain_slice(accum, out, c, s)

    return seg


@functools.lru_cache(maxsize=None)
def _make_edge_count():
    @functools.partial(
        pl.kernel,
        out_type=jax.ShapeDtypeStruct((2, NP, 16), jnp.float32),
        mesh=_sc_mesh(),
        scratch_types=[
            pltpu.VMEM((AW, EPB), jnp.int32),
            pltpu.VMEM((EPB, 16), jnp.float32),
            pltpu.VMEM_SHARED((NP, 16), jnp.float32),
            pltpu.SemaphoreType.DMA,
        ],
        compiler_params=pltpu.CompilerParams(use_tc_tiling_on_sc=False),
    )
    def cntk(eidxp, zeros16, ones_hbm, out, dst_v, ones_v, accum, s0):
        c = lax.axis_index("c")
        s = lax.axis_index("s")
        wid = c * 16 + s
        fb0 = wid * AW
        lim = jnp.minimum(AW, NBREAL - fb0)
        _zero_slice(zeros16, accum, s)
        pltpu.sync_copy(eidxp.at[1, pl.ds(fb0, AW)], dst_v)
        pltpu.sync_copy(ones_hbm, ones_v)
        plsc.subcore_barrier()

        @pl.loop(0, AW)
        def _(j):
            @pl.when(j < lim)
            def _():
                pltpu.async_copy(ones_v, accum.at[dst_v.at[j]], s0,
                                 add=True)

        @pl.loop(0, AW)
        def _(j):
            @pl.when(j < lim)
            def _():
                pltpu.make_async_copy(ones_v, accum.at[dst_v.at[j]],
                                      s0).wait()

        plsc.subcore_barrier()
        _drain_slice(accum, out, c, s)

    return cntk


def _tc_a_body(x_ref, w_ref, b_ref, o_ref):
    o_ref[...] = jnp.maximum(_dot(x_ref[...], w_ref[...]) + b_ref[...], 0.0)


def _tc_a(x, w, b):
    return pl.pallas_call(
        _tc_a_body,
        grid=(GRID,),
        in_specs=[
            pl.BlockSpec((BLK, D), lambda i: (i, 0)),
            pl.BlockSpec((D, D), lambda i: (0, 0)),
            pl.BlockSpec((1, D), lambda i: (0, 0)),
        ],
        out_specs=pl.BlockSpec((BLK, D), lambda i: (i, 0)),
        out_shape=jax.ShapeDtypeStruct((N, D), jnp.float32),
    )(x, w, b)


def _tc_b_body(x_ref, p_ref, c_ref, awx_ref, awm_ref, ab_ref, lw_ref, lb_ref,
               h1_ref, t_ref, inv_ref):
    p = p_ref[0] + p_ref[1]
    inv = 1.0 / jnp.maximum(c_ref[...], 1.0)
    mean = p * inv
    h = jnp.maximum(_dot(x_ref[...], awx_ref[...])
                    + _dot(mean, awm_ref[...]) + ab_ref[...], 0.0)
    nrm = jnp.sqrt(jnp.sum(h * h, axis=1, keepdims=True))
    h1 = h / jnp.maximum(nrm, 1e-12)
    h1_ref[...] = h1
    t_ref[...] = jnp.maximum(_dot(h1, lw_ref[...]) + lb_ref[...], 0.0)
    inv_ref[...] = inv


def _tc_b(x, partials, cnt_partials, awx, awm, ab, lw, lb):
    return pl.pallas_call(
        _tc_b_body,
        grid=(GRID,),
        in_specs=[
            pl.BlockSpec((BLK, D), lambda i: (i, 0)),
            pl.BlockSpec((2, BLK, D), lambda i: (0, i, 0)),
            pl.BlockSpec((BLK, 1), lambda i: (i, 0)),
            pl.BlockSpec((D, D), lambda i: (0, 0)),
            pl.BlockSpec((D, D), lambda i: (0, 0)),
            pl.BlockSpec((1, D), lambda i: (0, 0)),
            pl.BlockSpec((D, D), lambda i: (0, 0)),
            pl.BlockSpec((1, D), lambda i: (0, 0)),
        ],
        out_specs=[
            pl.BlockSpec((BLK, D), lambda i: (i, 0)),
            pl.BlockSpec((BLK, D), lambda i: (i, 0)),
            pl.BlockSpec((BLK, 1), lambda i: (i, 0)),
        ],
        out_shape=[
            jax.ShapeDtypeStruct((N, D), jnp.float32),
            jax.ShapeDtypeStruct((N, D), jnp.float32),
            jax.ShapeDtypeStruct((N, 1), jnp.float32),
        ],
    )(x, partials, cnt_partials, awx, awm, ab, lw, lb)


def _tc_c_body(h1_ref, p_ref, inv_ref, awx_ref, awm_ref, ab_ref,
               pw1_ref, pb1_ref, pw2_ref, pb2_ref, o_ref):
    mean = (p_ref[0] + p_ref[1]) * inv_ref[...]
    h = jnp.maximum(_dot(h1_ref[...], awx_ref[...])
                    + _dot(mean, awm_ref[...]) + ab_ref[...], 0.0)
    nrm = jnp.sqrt(jnp.sum(h * h, axis=1, keepdims=True))
    h2 = h / jnp.maximum(nrm, 1e-12)
    h3 = _dot(h2, pw1_ref[...]) + pb1_ref[...]
    z = _dot(h3, pw2_ref[...]) + pb2_ref[...]
    m = jnp.max(z, axis=1, keepdims=True)
    lse = m + jnp.log(jnp.sum(jnp.exp(z - m), axis=1, keepdims=True))
    o_ref[...] = (z - lse)[:, :O_DIM]


def _tc_c(h1, partials, inv, awx, awm, ab, pw1, pb1, pw2, pb2):
    return pl.pallas_call(
        _tc_c_body,
        grid=(GRID,),
        in_specs=[
            pl.BlockSpec((BLK, D), lambda i: (i, 0)),
            pl.BlockSpec((2, BLK, D), lambda i: (0, i, 0)),
            pl.BlockSpec((BLK, 1), lambda i: (i, 0)),
            pl.BlockSpec((D, D), lambda i: (0, 0)),
            pl.BlockSpec((D, D), lambda i: (0, 0)),
            pl.BlockSpec((1, D), lambda i: (0, 0)),
            pl.BlockSpec((D, D), lambda i: (0, 0)),
            pl.BlockSpec((1, D), lambda i: (0, 0)),
            pl.BlockSpec((D, D), lambda i: (0, 0)),
            pl.BlockSpec((1, D), lambda i: (0, 0)),
        ],
        out_specs=pl.BlockSpec((BLK, O_DIM), lambda i: (i, 0)),
        out_shape=jax.ShapeDtypeStruct((N, O_DIM), jnp.float32),
    )(h1, partials, inv, awx, awm, ab, pw1, pb1, pw2, pb2)


def kernel(x, edge_index, lin_W0, lin_b0, agg_W0, agg_b0,
           lin_W1, lin_b1, agg_W1, agg_b1,
           post_W1, post_b1, post_W2, post_b2):
    eidx = edge_index.astype(jnp.int32).reshape(2, NBREAL, EPB)
    eidxp = jnp.pad(eidx, ((0, 0), (0, NW * AW - NBREAL), (0, 0)))
    zeros128 = jnp.zeros((NP, D), jnp.float32)
    zeros16 = jnp.zeros((NP, 16), jnp.float32)
    ones16 = jnp.ones((EPB, 16), jnp.float32)

    lb0 = lin_b0.reshape(1, D)
    lb1 = lin_b1.reshape(1, D)
    ab0 = agg_b0.reshape(1, D)
    ab1 = agg_b1.reshape(1, D)
    pb1 = post_b1.reshape(1, D)
    pw2 = jnp.pad(post_W2, ((0, 0), (0, D - O_DIM)))
    pb2 = jnp.concatenate([post_b2,
                           jnp.full((D - O_DIM,), -1e30, jnp.float32)]
                          ).reshape(1, D)

    cntp = _make_edge_count()(eidxp, zeros16, ones16)
    cnt_col = cntp[0, :, 0:1] + cntp[1, :, 0:1]
    table0 = _tc_a(x, lin_W0, lb0)
    part0 = _make_seg_scatter()(table0, eidx, zeros128)
    h1, table1, inv = _tc_b(x, part0, cnt_col, agg_W0[:D], agg_W0[D:], ab0,
                            lin_W1, lb1)
    part1 = _make_seg_scatter()(table1, eidx, zeros128)
    return _tc_c(h1, part1, inv, agg_W1[:D], agg_W1[D:], ab1,
                 post_W1, pb1, pw2, pb2)
